# baseline (speedup 1.0000x reference)
import functools

import jax
import jax.numpy as jnp
from jax import lax
from jax.experimental import pallas as pl
from jax.experimental.pallas import tpu as pltpu
from jax.experimental.pallas import tpu_sc as plsc

F32 = jnp.float32
I32 = jnp.int32

T = 6
H = 4
C = 64
HC = H * C
B = 64
N = 10000
E = 30000
DPKG = 400
DE = 16
EPS = 1e-5

NW = 32
EC = 960
EP = NW * EC
CH = 96
NCG = EC // CH
EPW = EP // 16
NCS = EPW // CH
NA = 10240
RPT = NA // 16
AW = 136
BME = 1024
BMN = 400
BE = 600
TG = 2


def _mm(x, w, bm=BMN):
    TX, M, K = x.shape
    Tw, _, NN = w.shape

    def body(x_ref, w_ref, o_ref):
        o_ref[0] = jnp.dot(x_ref[0], w_ref[0], preferred_element_type=F32)

    return pl.pallas_call(
        body,
        grid=(Tw, M // bm),
        in_specs=[
            pl.BlockSpec((1, bm, K), lambda t, i: (t if TX > 1 else 0, i, 0)),
            pl.BlockSpec((1, K, NN), lambda t, i: (t, 0, 0)),
        ],
        out_specs=pl.BlockSpec((1, bm, NN), lambda t, i: (t, i, 0)),
        out_shape=jax.ShapeDtypeStruct((Tw, M, NN), F32),
    )(x, w)


def _pkg_proj(x_pkg, proj_W, proj_b, g, b):
    def body(x_ref, w_ref, pb_ref, g_ref, b_ref, o_ref):
        y = jnp.dot(x_ref[...], w_ref[...], preferred_element_type=F32)
        y = y + pb_ref[...]
        mu = jnp.mean(y, axis=-1, keepdims=True)
        var = jnp.mean((y - mu) ** 2, axis=-1, keepdims=True)
        y = (y - mu) / jnp.sqrt(var + EPS) * g_ref[...] + b_ref[...]
        o_ref[...] = jnp.maximum(y, 0.0)

    return pl.pallas_call(
        body,
        grid=(N // BMN,),
        in_specs=[
            pl.BlockSpec((BMN, DPKG), lambda i: (i, 0)),
            pl.BlockSpec((DPKG, HC), lambda i: (0, 0)),
            pl.BlockSpec((1, HC), lambda i: (0, 0)),
            pl.BlockSpec((1, HC), lambda i: (0, 0)),
            pl.BlockSpec((1, HC), lambda i: (0, 0)),
        ],
        out_specs=pl.BlockSpec((BMN, HC), lambda i: (i, 0)),
        out_shape=jax.ShapeDtypeStruct((N, HC), F32),
    )(x_pkg, proj_W, proj_b.reshape(1, HC), g.reshape(1, HC), b.reshape(1, HC))


def _logits(xle, xre, attm):
    TT = xle.shape[0]
    nb = EP // BME

    def body(xle_ref, xre_ref, attm_ref, lg_ref, mx_ref):
        i = pl.program_id(1)
        e = xle_ref[0] + xre_ref[0]
        e = jnp.where(e > 0, e, 0.2 * e)
        em = e * attm_ref[0]
        cols = [jnp.sum(em[:, h * C:(h + 1) * C], axis=1, keepdims=True)
                for h in range(H)]
        lg = jnp.concatenate(cols + [jnp.zeros((BME, 4), F32)], axis=1)
        lg_ref[0] = lg
        mxb = jnp.broadcast_to(jnp.max(lg, axis=0, keepdims=True), (8, 8))

        @pl.when(i == 0)
        def _():
            mx_ref[0] = mxb

        @pl.when(i > 0)
        def _():
            mx_ref[0] = jnp.maximum(mx_ref[0], mxb)

    return pl.pallas_call(
        body,
        grid=(TT, nb),
        in_specs=[
            pl.BlockSpec((1, BME, HC), lambda t, i: (t, i, 0)),
            pl.BlockSpec((1, BME, HC), lambda t, i: (t, i, 0)),
            pl.BlockSpec((1, 1, HC), lambda t, i: (t, 0, 0)),
        ],
        out_specs=[
            pl.BlockSpec((1, BME, 8), lambda t, i: (t, i, 0)),
            pl.BlockSpec((1, 8, 8), lambda t, i: (t, 0, 0)),
        ],
        out_shape=[
            jax.ShapeDtypeStruct((TT, EP, 8), F32),
            jax.ShapeDtypeStruct((TT, 8, 8), F32),
        ],
    )(xle, xre, attm)


def _exval(xle, lg, mx):
    TT = xle.shape[0]
    nb = EP // BME

    def body(xle_ref, lg_ref, mx_ref, o_ref):
        i = pl.program_id(1)
        gmax = mx_ref[0][0:1, :]
        ex = jnp.exp(lg_ref[0] - gmax)
        rows = lax.broadcasted_iota(I32, (BME, 8), 0) + i * BME
        ex = jnp.where(rows < E, ex, 0.0)
        exh = ex[:, :H].reshape(BME, H, 1)
        exb = jnp.broadcast_to(exh, (BME, H, C)).reshape(BME, HC)
        val = exb * xle_ref[0]
        z6 = jnp.zeros((BME, 6), F32)
        o_ref[0, 0] = jnp.concatenate([val[:, :128], ex[:, 0:2], z6], axis=1)
        o_ref[0, 1] = jnp.concatenate([val[:, 128:], ex[:, 2:4], z6], axis=1)

    return pl.pallas_call(
        body,
        grid=(TT, nb),
        in_specs=[
            pl.BlockSpec((1, BME, HC), lambda t, i: (t, i, 0)),
            pl.BlockSpec((1, BME, 8), lambda t, i: (t, i, 0)),
            pl.BlockSpec((1, 8, 8), lambda t, i: (t, 0, 0)),
        ],
        out_specs=pl.BlockSpec((1, 2, BME, AW), lambda t, i: (t, 0, i, 0)),
        out_shape=jax.ShapeDtypeStruct((TT, 2, EP, AW), F32),
    )(xle, lg, mx)


def _post(accout, g, b, wn_w=None):
    TT = accout.shape[0]
    nb = N // BMN

    def body(a_ref, g_ref, b_ref, *rest):
        if wn_w is None:
            (o_ref,) = rest
        else:
            wn_ref, o_ref, s_ref, smx_ref = rest
        num = jnp.concatenate([a_ref[0, 0][:, :128], a_ref[0, 1][:, :128]], axis=1)
        den = jnp.concatenate([a_ref[0, 0][:, 128:130], a_ref[0, 1][:, 128:130]], axis=1)
        denb = jnp.broadcast_to(den.reshape(BMN, H, 1), (BMN, H, C)).reshape(BMN, HC)
        y = num / (denb + 1e-16)
        mu = jnp.mean(y, axis=-1, keepdims=True)
        var = jnp.mean((y - mu) ** 2, axis=-1, keepdims=True)
        y = (y - mu) / jnp.sqrt(var + EPS) * g_ref[0] + b_ref[0]
        h = jnp.maximum(y, 0.0)
        o_ref[0] = h
        if wn_w is not None:
            i = pl.program_id(1)
            s = jnp.dot(h, wn_ref[...], preferred_element_type=F32)
            s_ref[0] = s
            smxb = jnp.broadcast_to(jnp.max(s, axis=0, keepdims=True), (8, 8))

            @pl.when(i == 0)
            def _():
                smx_ref[0] = smxb

            @pl.when(i > 0)
            def _():
                smx_ref[0] = jnp.maximum(smx_ref[0], smxb)

    in_specs = [
        pl.BlockSpec((1, 2, BMN, AW), lambda t, i: (t, 0, i, 0)),
        pl.BlockSpec((1, 1, HC), lambda t, i: (t, 0, 0)),
        pl.BlockSpec((1, 1, HC), lambda t, i: (t, 0, 0)),
    ]
    out_specs = [pl.BlockSpec((1, BMN, HC), lambda t, i: (t, i, 0))]
    out_shape = [jax.ShapeDtypeStruct((TT, N, HC), F32)]
    args = [accout, g.reshape(TT, 1, HC), b.reshape(TT, 1, HC)]
    if wn_w is not None:
        in_specs.append(pl.BlockSpec((HC, 8), lambda t, i: (0, 0)))
        out_specs += [
            pl.BlockSpec((1, BMN, 8), lambda t, i: (t, i, 0)),
            pl.BlockSpec((1, 8, 8), lambda t, i: (t, 0, 0)),
        ]
        out_shape += [
            jax.ShapeDtypeStruct((TT, N, 8), F32),
            jax.ShapeDtypeStruct((TT, 8, 8), F32),
        ]
        args.append(wn_w)

    res = pl.pallas_call(
        body, grid=(TT, nb), in_specs=in_specs,
        out_specs=out_specs, out_shape=out_shape,
    )(*args)
    return res if wn_w is not None else res[0]


def _npool(h2, s, smx, nb4):
    nb = N // BMN

    def body(h_ref, s_ref, smx_ref, nb_ref, o_ref):
        i = pl.program_id(1)
        gmax = smx_ref[0][0, 0]
        ex = jnp.exp(s_ref[0][:, 0:1] - gmax)
        seg = nb_ref[0, 0, 0]
        oh = (lax.broadcasted_iota(I32, (B, BMN), 0) == seg[None, :]).astype(F32)
        hx = jnp.concatenate(
            [h_ref[0] * ex, ex, jnp.zeros((BMN, 7), F32)], axis=1)
        hx_hi = hx.astype(jnp.bfloat16).astype(F32)
        hx_lo = hx - hx_hi
        contrib = (jnp.dot(oh, hx_hi, preferred_element_type=F32)
                   + jnp.dot(oh, hx_lo, preferred_element_type=F32))

        @pl.when(i == 0)
        def _():
            o_ref[0] = contrib

        @pl.when(i > 0)
        def _():
            o_ref[0] = o_ref[0] + contrib

    return pl.pallas_call(
        body,
        grid=(T, nb),
        in_specs=[
            pl.BlockSpec((1, BMN, HC), lambda t, i: (t, i, 0)),
            pl.BlockSpec((1, BMN, 8), lambda t, i: (t, i, 0)),
            pl.BlockSpec((1, 8, 8), lambda t, i: (t, 0, 0)),
            pl.BlockSpec((1, 1, 1, BMN), lambda t, i: (t, i, 0, 0)),
        ],
        out_specs=pl.BlockSpec((1, B, 264), lambda t, i: (t, 0, 0)),
        out_shape=jax.ShapeDtypeStruct((T, B, 264), F32),
    )(h2, s, smx, nb4)


def _escore(edge_attr, we_w):
    nb = E // BE

    def body(ea_ref, w_ref, s_ref, mx_ref):
        i = pl.program_id(1)
        s = jnp.dot(ea_ref[0], w_ref[...], preferred_element_type=F32)
        s_ref[0] = s
        mxb = jnp.broadcast_to(jnp.max(s, axis=0, keepdims=True), (8, 8))

        @pl.when(i == 0)
        def _():
            mx_ref[0] = mxb

        @pl.when(i > 0)
        def _():
            mx_ref[0] = jnp.maximum(mx_ref[0], mxb)

    return pl.pallas_call(
        body,
        grid=(T, nb),
        in_specs=[
            pl.BlockSpec((1, BE, DE), lambda t, i: (t, i, 0)),
            pl.BlockSpec((DE, 8), lambda t, i: (0, 0)),
        ],
        out_specs=[
            pl.BlockSpec((1, BE, 8), lambda t, i: (t, i, 0)),
            pl.BlockSpec((1, 8, 8), lambda t, i: (t, 0, 0)),
        ],
        out_shape=[
            jax.ShapeDtypeStruct((T, E, 8), F32),
            jax.ShapeDtypeStruct((T, 8, 8), F32),
        ],
    )(edge_attr, we_w)


def _epool(edge_attr, se, semx, eb4):
    nb = E // BE

    def body(ea_ref, s_ref, mx_ref, eb_ref, o_ref):
        i = pl.program_id(1)
        gmax = mx_ref[0][0, 0]
        ex = jnp.exp(s_ref[0][:, 0:1] - gmax)
        seg = eb_ref[0, 0, 0]
        oh = (lax.broadcasted_iota(I32, (B, BE), 0) == seg[None, :]).astype(F32)
        hx = jnp.concatenate(
            [ea_ref[0] * ex, ex, jnp.zeros((BE, 7), F32)], axis=1)
        hx_hi = hx.astype(jnp.bfloat16).astype(F32)
        hx_lo = hx - hx_hi
        contrib = (jnp.dot(oh, hx_hi, preferred_element_type=F32)
                   + jnp.dot(oh, hx_lo, preferred_element_type=F32))

        @pl.when(i == 0)
        def _():
            o_ref[0] = contrib

        @pl.when(i > 0)
        def _():
            o_ref[0] = o_ref[0] + contrib

    return pl.pallas_call(
        body,
        grid=(T, nb),
        in_specs=[
            pl.BlockSpec((1, BE, DE), lambda t, i: (t, i, 0)),
            pl.BlockSpec((1, BE, 8), lambda t, i: (t, i, 0)),
            pl.BlockSpec((1, 8, 8), lambda t, i: (t, 0, 0)),
            pl.BlockSpec((1, 1, 1, BE), lambda t, i: (t, i, 0, 0)),
        ],
        out_specs=pl.BlockSpec((1, B, 24), lambda t, i: (t, 0, 0)),
        out_shape=jax.ShapeDtypeStruct((T, B, 24), F32),
    )(edge_attr, se, semx, eb4)


def _final(Pn, Pe, Wcp, bc2):
    def body(pn_ref, pe_ref, w_ref, bc_ref, o_ref):
        npool = jnp.zeros((B, HC), F32)
        epool = jnp.zeros((B, DE), F32)
        for t in range(T):
            npool = npool + pn_ref[t][:, :HC] / (pn_ref[t][:, HC:HC + 1] + 1e-16)
            epool = epool + pe_ref[t][:, :DE] / (pe_ref[t][:, DE:DE + 1] + 1e-16)
        g = jnp.concatenate([npool * (1.0 / T), epool * (1.0 / T)], axis=1)
        o_ref[...] = jnp.dot(g, w_ref[...], preferred_element_type=F32) + bc_ref[0, 0]

    return pl.pallas_call(
        body,
        grid=(1,),
        in_specs=[
            pl.BlockSpec((T, B, 264), lambda i: (0, 0, 0)),
            pl.BlockSpec((T, B, 24), lambda i: (0, 0, 0)),
            pl.BlockSpec((HC + DE, 128), lambda i: (0, 0)),
            pl.BlockSpec((1, 1), lambda i: (0, 0)),
        ],
        out_specs=pl.BlockSpec((B, 128), lambda i: (0, 0)),
        out_shape=jax.ShapeDtypeStruct((B, 128), F32),
    )(Pn, Pe, Wcp, bc2)


def _sc_gather(xl_flat, xr_flat, si_r, di_r):
    TT = si_r.shape[1] // NCG
    mesh = plsc.VectorSubcoreMesh(core_axis_name="c", subcore_axis_name="s", num_cores=2, num_subcores=16)

    @functools.partial(
        pl.kernel,
        out_type=(
            jax.ShapeDtypeStruct((TT, EP, HC), F32),
            jax.ShapeDtypeStruct((TT, EP, HC), F32),
        ),
        mesh=mesh,
        scratch_types=[
            pltpu.VMEM((TT * NCG, CH), I32),
            pltpu.VMEM((TT * NCG, CH), I32),
            pltpu.VMEM((CH, HC), F32), pltpu.VMEM((CH, HC), F32),
            pltpu.VMEM((CH, HC), F32), pltpu.VMEM((CH, HC), F32),
            pltpu.SemaphoreType.DMA, pltpu.SemaphoreType.DMA,
            pltpu.SemaphoreType.DMA, pltpu.SemaphoreType.DMA,
            pltpu.SemaphoreType.DMA, pltpu.SemaphoreType.DMA,
            pltpu.SemaphoreType.DMA, pltpu.SemaphoreType.DMA,
        ],
        compiler_params=pltpu.CompilerParams(use_tc_tiling_on_sc=False),
    )
    def k(xl_hbm, xr_hbm, si_hbm, di_hbm, xle_hbm, xre_hbm,
          sidx_v, didx_v, l0, l1, r0, r1,
          gl0, gl1, gr0, gr1, wl0, wl1, wr0, wr1):
        c = lax.axis_index("c")
        s = lax.axis_index("s")
        wid = s * 2 + c
        L = (l0, l1)
        R = (r0, r1)
        GL = (gl0, gl1)
        GR = (gr0, gr1)
        WL = (wl0, wl1)
        WR = (wr0, wr1)
        pltpu.sync_copy(si_hbm.at[wid], sidx_v)
        pltpu.sync_copy(di_hbm.at[wid], didx_v)

        def body_t(t, carry):
            gdesc = [None, None]
            wdesc = [None, None]

            def gat(j, b):
                row = t * NCG + j
                dl = pltpu.async_copy(xl_hbm.at[sidx_v.at[row]], L[b], GL[b])
                dr = pltpu.async_copy(xr_hbm.at[didx_v.at[row]], R[b], GR[b])
                gdesc[b] = (dl, dr)

            def wb(j, b):
                base = wid * EC + j * CH
                dl = pltpu.async_copy(L[b], xle_hbm.at[t, pl.ds(base, CH), :], WL[b])
                dr = pltpu.async_copy(R[b], xre_hbm.at[t, pl.ds(base, CH), :], WR[b])
                wdesc[b] = (dl, dr)

            gat(0, 0)
            for j in range(1, NCG):
                b = j % 2
                pb = 1 - b
                if j >= 2:
                    wdesc[b][0].wait()
                    wdesc[b][1].wait()
                gat(j, b)
                gdesc[pb][0].wait()
                gdesc[pb][1].wait()
                wb(j - 1, pb)
            lb = (NCG - 1) % 2
            gdesc[lb][0].wait()
            gdesc[lb][1].wait()
            wb(NCG - 1, lb)
            for b in range(2):
                wdesc[b][0].wait()
                wdesc[b][1].wait()
            return carry

        lax.fori_loop(0, TT, body_t, 0)

    return k(xl_flat, xr_flat, si_r, di_r)


def _sc_scatter(vaug, di3, zrows):
    TT = vaug.shape[0]
    mesh = plsc.VectorSubcoreMesh(core_axis_name="c", subcore_axis_name="s", num_cores=2, num_subcores=16)

    @functools.partial(
        pl.kernel,
        out_type=jax.ShapeDtypeStruct((TT, 2, NA, AW), F32),
        mesh=mesh,
        scratch_types=[
            pltpu.VMEM((TT * NCS, CH), I32),
            pltpu.VMEM((CH, AW), F32), pltpu.VMEM((CH, AW), F32),
            pltpu.VMEM_SHARED((NA, AW), F32),
            pltpu.SemaphoreType.DMA, pltpu.SemaphoreType.DMA,
            pltpu.SemaphoreType.DMA, pltpu.SemaphoreType.DMA,
        ],
        compiler_params=pltpu.CompilerParams(use_tc_tiling_on_sc=False),
    )
    def k(vaug_hbm, di_hbm, z_hbm, accout_hbm, didx_v, v0, v1, acc_sh,
          vs0, vs1, ss0, ss1):
        c = lax.axis_index("c")
        s = lax.axis_index("s")
        rbase = s * RPT
        V = (v0, v1)
        VS = (vs0, vs1)
        SS = (ss0, ss1)
        pltpu.sync_copy(di_hbm.at[s], didx_v)
        pltpu.sync_copy(z_hbm, acc_sh.at[pl.ds(rbase, RPT)])
        plsc.subcore_barrier()

        def body_t(t, carry):
            vdesc = [None, None]
            sdesc = [None, None]

            def vload(j, b):
                ebase = s * EPW + j * CH
                vdesc[b] = pltpu.async_copy(
                    vaug_hbm.at[t, c, pl.ds(ebase, CH), :], V[b], VS[b])

            def scat(j, b):
                row = t * NCS + j
                sdesc[b] = pltpu.async_copy(
                    V[b], acc_sh.at[didx_v.at[row]], SS[b], add=True)

            vload(0, 0)
            for j in range(1, NCS):
                b = j % 2
                pb = 1 - b
                if j >= 2:
                    sdesc[b].wait()
                vload(j, b)
                vdesc[pb].wait()
                scat(j - 1, pb)
            lb = (NCS - 1) % 2
            vdesc[lb].wait()
            scat(NCS - 1, lb)
            sdesc[0].wait()
            sdesc[1].wait()
            plsc.subcore_barrier()
            pltpu.sync_copy(acc_sh.at[pl.ds(rbase, RPT)],
                            accout_hbm.at[t, c, pl.ds(rbase, RPT), :])
            pltpu.sync_copy(z_hbm, acc_sh.at[pl.ds(rbase, RPT)])
            plsc.subcore_barrier()
            return carry

        lax.fori_loop(0, TT, body_t, 0)

    return k(vaug, di3, zrows)


def _head_mask():
    head_col = jnp.repeat(jnp.arange(H, dtype=I32), C)
    return (head_col[:, None] == jnp.arange(8, dtype=I32)[None, :]).astype(F32)


def kernel(x_pkg, x_tgt, edge_attr, c1_Wl, c1_Wr, c1_att, c2_Wl, c2_Wr, c2_att,
           ln1_g, ln1_b, ln2_g, ln2_b, ln1p_g, ln1p_b, proj_W, proj_b,
           wn_att, we_att, Wc, bc, edge_index, node_batch, edge_batch):
    src = edge_index[:, 0].astype(I32)
    dst = edge_index[:, 1].astype(I32)
    pad = jnp.zeros((T, EP - E), I32)
    srcp = jnp.concatenate([src, pad], axis=1)
    dstp = jnp.concatenate([dst, pad], axis=1)
    toff = (jnp.arange(T, dtype=I32) * N)[:, None]
    si_off = srcp + toff
    di_off = dstp + toff

    attm1 = c1_att.reshape(T, 1, HC)
    attm2 = c2_att.reshape(T, 1, HC)
    wn_w = jnp.zeros((HC, 8), F32).at[:, 0].set(wn_att)
    we_w = jnp.zeros((DE, 8), F32).at[:, 0].set(we_att)
    Wcp = jnp.zeros((HC + DE, 128), F32).at[:, 0].set(Wc[:, 0])
    bc2 = bc.reshape(1, 1)
    zrows = jnp.zeros((RPT, AW), F32)
    nb4 = node_batch.astype(I32).reshape(T, N // BMN, 1, BMN)
    eb4 = edge_batch.astype(I32).reshape(T, E // BE, 1, BE)

    xl1 = _mm(x_pkg[None], c1_Wl)
    xr1 = _mm(x_tgt, c1_Wr)
    pkg = _pkg_proj(x_pkg, proj_W, proj_b, ln1p_g, ln1p_b)
    xl2 = _mm(pkg[None], c2_Wl)
    xl1f = xl1.reshape(T * N, HC)
    xr1f = xr1.reshape(T * N, HC)
    xl2f = xl2.reshape(T * N, HC)

    def reord_g(a):
        TT = a.shape[0]
        return a.reshape(TT, NW, NCG, CH).transpose(1, 0, 2, 3).reshape(
            NW, TT * NCG, CH)

    def reord_s(a):
        TT = a.shape[0]
        return a.reshape(TT, 16, NCS, CH).transpose(1, 0, 2, 3).reshape(
            16, TT * NCS, CH)

    groups = [(0, 2), (2, 4), (4, 6)]
    toff_g = (jnp.arange(TG, dtype=I32) * N)[:, None]

    sir_g = [reord_g(si_off[g0:g1]) for g0, g1 in groups]
    dir_g = [reord_g(di_off[g0:g1]) for g0, g1 in groups]
    di3_g = [reord_s(dstp[g0:g1]) for g0, g1 in groups]
    dil_g = [reord_g(dstp[g0:g1] + toff_g) for g0, g1 in groups]

    h1_parts = []
    for i, (g0, g1) in enumerate(groups):
        xle, xre = _sc_gather(xl1f, xr1f, sir_g[i], dir_g[i])
        lg, mx = _logits(xle, xre, attm1[g0:g1])
        vaug = _exval(xle, lg, mx)
        accout = _sc_scatter(vaug, di3_g[i], zrows)
        h1_parts.append(_post(accout, ln1_g[g0:g1], ln1_b[g0:g1]))

    h2_parts, s_parts, smx_parts = [], [], []
    for i, (g0, g1) in enumerate(groups):
        xr2 = _mm(h1_parts[i], c2_Wr[g0:g1])
        xle2, xre2 = _sc_gather(xl2f, xr2.reshape(TG * N, HC),
                                sir_g[i], dil_g[i])
        lg2, mx2 = _logits(xle2, xre2, attm2[g0:g1])
        vaug2 = _exval(xle2, lg2, mx2)
        accout2 = _sc_scatter(vaug2, di3_g[i], zrows)
        h2g, sg, smxg = _post(accout2, ln2_g[g0:g1], ln2_b[g0:g1], wn_w)
        h2_parts.append(h2g)
        s_parts.append(sg)
        smx_parts.append(smxg)

    h2 = jnp.concatenate(h2_parts, axis=0)
    s = jnp.concatenate(s_parts, axis=0)
    smx = jnp.concatenate(smx_parts, axis=0)

    Pn = _npool(h2, s, smx, nb4)
    se, semx = _escore(edge_attr, we_w)
    Pe = _epool(edge_attr, se, semx, eb4)
    out = _final(Pn, Pe, Wcp, bc2)
    return out[:, 0]


# speedup vs baseline: 1.0097x; 1.0097x over previous
import functools

import jax
import jax.numpy as jnp
from jax import lax
from jax.experimental import pallas as pl
from jax.experimental.pallas import tpu as pltpu
from jax.experimental.pallas import tpu_sc as plsc

F32 = jnp.float32
I32 = jnp.int32

T = 6
H = 4
C = 64
HC = H * C
B = 64
N = 10000
E = 30000
DPKG = 400
DE = 16
EPS = 1e-5

NW = 32
EC = 960
EP = NW * EC
CH = 96
NCG = EC // CH
EPW = EP // 16
NCS = EPW // CH
NA = 10240
RPT = NA // 16
AW = 136
BME = 1024
BMN = 400
BE = 600
TG = 1


def _mm(x, w, bm=BMN):
    TX, M, K = x.shape
    Tw, _, NN = w.shape

    def body(x_ref, w_ref, o_ref):
        o_ref[0] = jnp.dot(x_ref[0], w_ref[0], preferred_element_type=F32)

    return pl.pallas_call(
        body,
        grid=(Tw, M // bm),
        in_specs=[
            pl.BlockSpec((1, bm, K), lambda t, i: (t if TX > 1 else 0, i, 0)),
            pl.BlockSpec((1, K, NN), lambda t, i: (t, 0, 0)),
        ],
        out_specs=pl.BlockSpec((1, bm, NN), lambda t, i: (t, i, 0)),
        out_shape=jax.ShapeDtypeStruct((Tw, M, NN), F32),
    )(x, w)


def _pkg_proj(x_pkg, proj_W, proj_b, g, b):
    def body(x_ref, w_ref, pb_ref, g_ref, b_ref, o_ref):
        y = jnp.dot(x_ref[...], w_ref[...], preferred_element_type=F32)
        y = y + pb_ref[...]
        mu = jnp.mean(y, axis=-1, keepdims=True)
        var = jnp.mean((y - mu) ** 2, axis=-1, keepdims=True)
        y = (y - mu) / jnp.sqrt(var + EPS) * g_ref[...] + b_ref[...]
        o_ref[...] = jnp.maximum(y, 0.0)

    return pl.pallas_call(
        body,
        grid=(N // BMN,),
        in_specs=[
            pl.BlockSpec((BMN, DPKG), lambda i: (i, 0)),
            pl.BlockSpec((DPKG, HC), lambda i: (0, 0)),
            pl.BlockSpec((1, HC), lambda i: (0, 0)),
            pl.BlockSpec((1, HC), lambda i: (0, 0)),
            pl.BlockSpec((1, HC), lambda i: (0, 0)),
        ],
        out_specs=pl.BlockSpec((BMN, HC), lambda i: (i, 0)),
        out_shape=jax.ShapeDtypeStruct((N, HC), F32),
    )(x_pkg, proj_W, proj_b.reshape(1, HC), g.reshape(1, HC), b.reshape(1, HC))


def _logits(xle, xre, attm):
    TT = xle.shape[0]
    nb = EP // BME

    def body(xle_ref, xre_ref, attm_ref, lg_ref, mx_ref):
        i = pl.program_id(1)
        e = xle_ref[0] + xre_ref[0]
        e = jnp.where(e > 0, e, 0.2 * e)
        em = e * attm_ref[0]
        cols = [jnp.sum(em[:, h * C:(h + 1) * C], axis=1, keepdims=True)
                for h in range(H)]
        lg = jnp.concatenate(cols + [jnp.zeros((BME, 4), F32)], axis=1)
        lg_ref[0] = lg
        mxb = jnp.broadcast_to(jnp.max(lg, axis=0, keepdims=True), (8, 8))

        @pl.when(i == 0)
        def _():
            mx_ref[0] = mxb

        @pl.when(i > 0)
        def _():
            mx_ref[0] = jnp.maximum(mx_ref[0], mxb)

    return pl.pallas_call(
        body,
        grid=(TT, nb),
        in_specs=[
            pl.BlockSpec((1, BME, HC), lambda t, i: (t, i, 0)),
            pl.BlockSpec((1, BME, HC), lambda t, i: (t, i, 0)),
            pl.BlockSpec((1, 1, HC), lambda t, i: (t, 0, 0)),
        ],
        out_specs=[
            pl.BlockSpec((1, BME, 8), lambda t, i: (t, i, 0)),
            pl.BlockSpec((1, 8, 8), lambda t, i: (t, 0, 0)),
        ],
        out_shape=[
            jax.ShapeDtypeStruct((TT, EP, 8), F32),
            jax.ShapeDtypeStruct((TT, 8, 8), F32),
        ],
    )(xle, xre, attm)


def _exval(xle, lg, mx):
    TT = xle.shape[0]
    nb = EP // BME

    def body(xle_ref, lg_ref, mx_ref, o_ref):
        i = pl.program_id(1)
        gmax = mx_ref[0][0:1, :]
        ex = jnp.exp(lg_ref[0] - gmax)
        rows = lax.broadcasted_iota(I32, (BME, 8), 0) + i * BME
        ex = jnp.where(rows < E, ex, 0.0)
        exh = ex[:, :H].reshape(BME, H, 1)
        exb = jnp.broadcast_to(exh, (BME, H, C)).reshape(BME, HC)
        val = exb * xle_ref[0]
        z6 = jnp.zeros((BME, 6), F32)
        o_ref[0, 0] = jnp.concatenate([val[:, :128], ex[:, 0:2], z6], axis=1)
        o_ref[0, 1] = jnp.concatenate([val[:, 128:], ex[:, 2:4], z6], axis=1)

    return pl.pallas_call(
        body,
        grid=(TT, nb),
        in_specs=[
            pl.BlockSpec((1, BME, HC), lambda t, i: (t, i, 0)),
            pl.BlockSpec((1, BME, 8), lambda t, i: (t, i, 0)),
            pl.BlockSpec((1, 8, 8), lambda t, i: (t, 0, 0)),
        ],
        out_specs=pl.BlockSpec((1, 2, BME, AW), lambda t, i: (t, 0, i, 0)),
        out_shape=jax.ShapeDtypeStruct((TT, 2, EP, AW), F32),
    )(xle, lg, mx)


def _post(accout, g, b, wn_w=None):
    TT = accout.shape[0]
    nb = N // BMN

    def body(a_ref, g_ref, b_ref, *rest):
        if wn_w is None:
            (o_ref,) = rest
        else:
            wn_ref, o_ref, s_ref, smx_ref = rest
        num = jnp.concatenate([a_ref[0, 0][:, :128], a_ref[0, 1][:, :128]], axis=1)
        den = jnp.concatenate([a_ref[0, 0][:, 128:130], a_ref[0, 1][:, 128:130]], axis=1)
        denb = jnp.broadcast_to(den.reshape(BMN, H, 1), (BMN, H, C)).reshape(BMN, HC)
        y = num / (denb + 1e-16)
        mu = jnp.mean(y, axis=-1, keepdims=True)
        var = jnp.mean((y - mu) ** 2, axis=-1, keepdims=True)
        y = (y - mu) / jnp.sqrt(var + EPS) * g_ref[0] + b_ref[0]
        h = jnp.maximum(y, 0.0)
        o_ref[0] = h
        if wn_w is not None:
            i = pl.program_id(1)
            s = jnp.dot(h, wn_ref[...], preferred_element_type=F32)
            s_ref[0] = s
            smxb = jnp.broadcast_to(jnp.max(s, axis=0, keepdims=True), (8, 8))

            @pl.when(i == 0)
            def _():
                smx_ref[0] = smxb

            @pl.when(i > 0)
            def _():
                smx_ref[0] = jnp.maximum(smx_ref[0], smxb)

    in_specs = [
        pl.BlockSpec((1, 2, BMN, AW), lambda t, i: (t, 0, i, 0)),
        pl.BlockSpec((1, 1, HC), lambda t, i: (t, 0, 0)),
        pl.BlockSpec((1, 1, HC), lambda t, i: (t, 0, 0)),
    ]
    out_specs = [pl.BlockSpec((1, BMN, HC), lambda t, i: (t, i, 0))]
    out_shape = [jax.ShapeDtypeStruct((TT, N, HC), F32)]
    args = [accout, g.reshape(TT, 1, HC), b.reshape(TT, 1, HC)]
    if wn_w is not None:
        in_specs.append(pl.BlockSpec((HC, 8), lambda t, i: (0, 0)))
        out_specs += [
            pl.BlockSpec((1, BMN, 8), lambda t, i: (t, i, 0)),
            pl.BlockSpec((1, 8, 8), lambda t, i: (t, 0, 0)),
        ]
        out_shape += [
            jax.ShapeDtypeStruct((TT, N, 8), F32),
            jax.ShapeDtypeStruct((TT, 8, 8), F32),
        ]
        args.append(wn_w)

    res = pl.pallas_call(
        body, grid=(TT, nb), in_specs=in_specs,
        out_specs=out_specs, out_shape=out_shape,
    )(*args)
    return res if wn_w is not None else res[0]


def _npool(h2, s, smx, nb4):
    nb = N // BMN

    def body(h_ref, s_ref, smx_ref, nb_ref, o_ref):
        i = pl.program_id(1)
        gmax = smx_ref[0][0, 0]
        ex = jnp.exp(s_ref[0][:, 0:1] - gmax)
        seg = nb_ref[0, 0, 0]
        oh = (lax.broadcasted_iota(I32, (B, BMN), 0) == seg[None, :]).astype(F32)
        hx = jnp.concatenate(
            [h_ref[0] * ex, ex, jnp.zeros((BMN, 7), F32)], axis=1)
        hx_hi = hx.astype(jnp.bfloat16).astype(F32)
        hx_lo = hx - hx_hi
        contrib = (jnp.dot(oh, hx_hi, preferred_element_type=F32)
                   + jnp.dot(oh, hx_lo, preferred_element_type=F32))

        @pl.when(i == 0)
        def _():
            o_ref[0] = contrib

        @pl.when(i > 0)
        def _():
            o_ref[0] = o_ref[0] + contrib

    return pl.pallas_call(
        body,
        grid=(T, nb),
        in_specs=[
            pl.BlockSpec((1, BMN, HC), lambda t, i: (t, i, 0)),
            pl.BlockSpec((1, BMN, 8), lambda t, i: (t, i, 0)),
            pl.BlockSpec((1, 8, 8), lambda t, i: (t, 0, 0)),
            pl.BlockSpec((1, 1, 1, BMN), lambda t, i: (t, i, 0, 0)),
        ],
        out_specs=pl.BlockSpec((1, B, 264), lambda t, i: (t, 0, 0)),
        out_shape=jax.ShapeDtypeStruct((T, B, 264), F32),
    )(h2, s, smx, nb4)


def _escore(edge_attr, we_w):
    nb = E // BE

    def body(ea_ref, w_ref, s_ref, mx_ref):
        i = pl.program_id(1)
        s = jnp.dot(ea_ref[0], w_ref[...], preferred_element_type=F32)
        s_ref[0] = s
        mxb = jnp.broadcast_to(jnp.max(s, axis=0, keepdims=True), (8, 8))

        @pl.when(i == 0)
        def _():
            mx_ref[0] = mxb

        @pl.when(i > 0)
        def _():
            mx_ref[0] = jnp.maximum(mx_ref[0], mxb)

    return pl.pallas_call(
        body,
        grid=(T, nb),
        in_specs=[
            pl.BlockSpec((1, BE, DE), lambda t, i: (t, i, 0)),
            pl.BlockSpec((DE, 8), lambda t, i: (0, 0)),
        ],
        out_specs=[
            pl.BlockSpec((1, BE, 8), lambda t, i: (t, i, 0)),
            pl.BlockSpec((1, 8, 8), lambda t, i: (t, 0, 0)),
        ],
        out_shape=[
            jax.ShapeDtypeStruct((T, E, 8), F32),
            jax.ShapeDtypeStruct((T, 8, 8), F32),
        ],
    )(edge_attr, we_w)


def _epool(edge_attr, se, semx, eb4):
    nb = E // BE

    def body(ea_ref, s_ref, mx_ref, eb_ref, o_ref):
        i = pl.program_id(1)
        gmax = mx_ref[0][0, 0]
        ex = jnp.exp(s_ref[0][:, 0:1] - gmax)
        seg = eb_ref[0, 0, 0]
        oh = (lax.broadcasted_iota(I32, (B, BE), 0) == seg[None, :]).astype(F32)
        hx = jnp.concatenate(
            [ea_ref[0] * ex, ex, jnp.zeros((BE, 7), F32)], axis=1)
        hx_hi = hx.astype(jnp.bfloat16).astype(F32)
        hx_lo = hx - hx_hi
        contrib = (jnp.dot(oh, hx_hi, preferred_element_type=F32)
                   + jnp.dot(oh, hx_lo, preferred_element_type=F32))

        @pl.when(i == 0)
        def _():
            o_ref[0] = contrib

        @pl.when(i > 0)
        def _():
            o_ref[0] = o_ref[0] + contrib

    return pl.pallas_call(
        body,
        grid=(T, nb),
        in_specs=[
            pl.BlockSpec((1, BE, DE), lambda t, i: (t, i, 0)),
            pl.BlockSpec((1, BE, 8), lambda t, i: (t, i, 0)),
            pl.BlockSpec((1, 8, 8), lambda t, i: (t, 0, 0)),
            pl.BlockSpec((1, 1, 1, BE), lambda t, i: (t, i, 0, 0)),
        ],
        out_specs=pl.BlockSpec((1, B, 24), lambda t, i: (t, 0, 0)),
        out_shape=jax.ShapeDtypeStruct((T, B, 24), F32),
    )(edge_attr, se, semx, eb4)


def _final(Pn, Pe, Wcp, bc2):
    def body(pn_ref, pe_ref, w_ref, bc_ref, o_ref):
        npool = jnp.zeros((B, HC), F32)
        epool = jnp.zeros((B, DE), F32)
        for t in range(T):
            npool = npool + pn_ref[t][:, :HC] / (pn_ref[t][:, HC:HC + 1] + 1e-16)
            epool = epool + pe_ref[t][:, :DE] / (pe_ref[t][:, DE:DE + 1] + 1e-16)
        g = jnp.concatenate([npool * (1.0 / T), epool * (1.0 / T)], axis=1)
        o_ref[...] = jnp.dot(g, w_ref[...], preferred_element_type=F32) + bc_ref[0, 0]

    return pl.pallas_call(
        body,
        grid=(1,),
        in_specs=[
            pl.BlockSpec((T, B, 264), lambda i: (0, 0, 0)),
            pl.BlockSpec((T, B, 24), lambda i: (0, 0, 0)),
            pl.BlockSpec((HC + DE, 128), lambda i: (0, 0)),
            pl.BlockSpec((1, 1), lambda i: (0, 0)),
        ],
        out_specs=pl.BlockSpec((B, 128), lambda i: (0, 0)),
        out_shape=jax.ShapeDtypeStruct((B, 128), F32),
    )(Pn, Pe, Wcp, bc2)


def _sc_gather(xl_flat, xr_flat, si_r, di_r):
    TT = si_r.shape[1] // NCG
    mesh = plsc.VectorSubcoreMesh(core_axis_name="c", subcore_axis_name="s", num_cores=2, num_subcores=16)

    @functools.partial(
        pl.kernel,
        out_type=(
            jax.ShapeDtypeStruct((TT, EP, HC), F32),
            jax.ShapeDtypeStruct((TT, EP, HC), F32),
        ),
        mesh=mesh,
        scratch_types=[
            pltpu.VMEM((TT * NCG, CH), I32),
            pltpu.VMEM((TT * NCG, CH), I32),
            pltpu.VMEM((CH, HC), F32), pltpu.VMEM((CH, HC), F32),
            pltpu.VMEM((CH, HC), F32), pltpu.VMEM((CH, HC), F32),
            pltpu.SemaphoreType.DMA, pltpu.SemaphoreType.DMA,
            pltpu.SemaphoreType.DMA, pltpu.SemaphoreType.DMA,
            pltpu.SemaphoreType.DMA, pltpu.SemaphoreType.DMA,
            pltpu.SemaphoreType.DMA, pltpu.SemaphoreType.DMA,
        ],
        compiler_params=pltpu.CompilerParams(use_tc_tiling_on_sc=False),
    )
    def k(xl_hbm, xr_hbm, si_hbm, di_hbm, xle_hbm, xre_hbm,
          sidx_v, didx_v, l0, l1, r0, r1,
          gl0, gl1, gr0, gr1, wl0, wl1, wr0, wr1):
        c = lax.axis_index("c")
        s = lax.axis_index("s")
        wid = s * 2 + c
        L = (l0, l1)
        R = (r0, r1)
        GL = (gl0, gl1)
        GR = (gr0, gr1)
        WL = (wl0, wl1)
        WR = (wr0, wr1)
        pltpu.sync_copy(si_hbm.at[wid], sidx_v)
        pltpu.sync_copy(di_hbm.at[wid], didx_v)

        def body_t(t, carry):
            gdesc = [None, None]
            wdesc = [None, None]

            def gat(j, b):
                row = t * NCG + j
                dl = pltpu.async_copy(xl_hbm.at[sidx_v.at[row]], L[b], GL[b])
                dr = pltpu.async_copy(xr_hbm.at[didx_v.at[row]], R[b], GR[b])
                gdesc[b] = (dl, dr)

            def wb(j, b):
                base = wid * EC + j * CH
                dl = pltpu.async_copy(L[b], xle_hbm.at[t, pl.ds(base, CH), :], WL[b])
                dr = pltpu.async_copy(R[b], xre_hbm.at[t, pl.ds(base, CH), :], WR[b])
                wdesc[b] = (dl, dr)

            gat(0, 0)
            for j in range(1, NCG):
                b = j % 2
                pb = 1 - b
                if j >= 2:
                    wdesc[b][0].wait()
                    wdesc[b][1].wait()
                gat(j, b)
                gdesc[pb][0].wait()
                gdesc[pb][1].wait()
                wb(j - 1, pb)
            lb = (NCG - 1) % 2
            gdesc[lb][0].wait()
            gdesc[lb][1].wait()
            wb(NCG - 1, lb)
            for b in range(2):
                wdesc[b][0].wait()
                wdesc[b][1].wait()
            return carry

        lax.fori_loop(0, TT, body_t, 0)

    return k(xl_flat, xr_flat, si_r, di_r)


def _sc_scatter(vaug, di3, zrows):
    TT = vaug.shape[0]
    mesh = plsc.VectorSubcoreMesh(core_axis_name="c", subcore_axis_name="s", num_cores=2, num_subcores=16)

    @functools.partial(
        pl.kernel,
        out_type=jax.ShapeDtypeStruct((TT, 2, NA, AW), F32),
        mesh=mesh,
        scratch_types=[
            pltpu.VMEM((TT * NCS, CH), I32),
            pltpu.VMEM((CH, AW), F32), pltpu.VMEM((CH, AW), F32),
            pltpu.VMEM_SHARED((NA, AW), F32),
            pltpu.SemaphoreType.DMA, pltpu.SemaphoreType.DMA,
            pltpu.SemaphoreType.DMA, pltpu.SemaphoreType.DMA,
        ],
        compiler_params=pltpu.CompilerParams(use_tc_tiling_on_sc=False),
    )
    def k(vaug_hbm, di_hbm, z_hbm, accout_hbm, didx_v, v0, v1, acc_sh,
          vs0, vs1, ss0, ss1):
        c = lax.axis_index("c")
        s = lax.axis_index("s")
        rbase = s * RPT
        V = (v0, v1)
        VS = (vs0, vs1)
        SS = (ss0, ss1)
        pltpu.sync_copy(di_hbm.at[s], didx_v)
        pltpu.sync_copy(z_hbm, acc_sh.at[pl.ds(rbase, RPT)])
        plsc.subcore_barrier()

        def body_t(t, carry):
            vdesc = [None, None]
            sdesc = [None, None]

            def vload(j, b):
                ebase = s * EPW + j * CH
                vdesc[b] = pltpu.async_copy(
                    vaug_hbm.at[t, c, pl.ds(ebase, CH), :], V[b], VS[b])

            def scat(j, b):
                row = t * NCS + j
                sdesc[b] = pltpu.async_copy(
                    V[b], acc_sh.at[didx_v.at[row]], SS[b], add=True)

            vload(0, 0)
            for j in range(1, NCS):
                b = j % 2
                pb = 1 - b
                if j >= 2:
                    sdesc[b].wait()
                vload(j, b)
                vdesc[pb].wait()
                scat(j - 1, pb)
            lb = (NCS - 1) % 2
            vdesc[lb].wait()
            scat(NCS - 1, lb)
            sdesc[0].wait()
            sdesc[1].wait()
            plsc.subcore_barrier()
            pltpu.sync_copy(acc_sh.at[pl.ds(rbase, RPT)],
                            accout_hbm.at[t, c, pl.ds(rbase, RPT), :])
            pltpu.sync_copy(z_hbm, acc_sh.at[pl.ds(rbase, RPT)])
            plsc.subcore_barrier()
            return carry

        lax.fori_loop(0, TT, body_t, 0)

    return k(vaug, di3, zrows)


def _head_mask():
    head_col = jnp.repeat(jnp.arange(H, dtype=I32), C)
    return (head_col[:, None] == jnp.arange(8, dtype=I32)[None, :]).astype(F32)


def kernel(x_pkg, x_tgt, edge_attr, c1_Wl, c1_Wr, c1_att, c2_Wl, c2_Wr, c2_att,
           ln1_g, ln1_b, ln2_g, ln2_b, ln1p_g, ln1p_b, proj_W, proj_b,
           wn_att, we_att, Wc, bc, edge_index, node_batch, edge_batch):
    src = edge_index[:, 0].astype(I32)
    dst = edge_index[:, 1].astype(I32)
    pad = jnp.zeros((T, EP - E), I32)
    srcp = jnp.concatenate([src, pad], axis=1)
    dstp = jnp.concatenate([dst, pad], axis=1)
    toff = (jnp.arange(T, dtype=I32) * N)[:, None]
    si_off = srcp + toff
    di_off = dstp + toff

    attm1 = c1_att.reshape(T, 1, HC)
    attm2 = c2_att.reshape(T, 1, HC)
    wn_w = jnp.zeros((HC, 8), F32).at[:, 0].set(wn_att)
    we_w = jnp.zeros((DE, 8), F32).at[:, 0].set(we_att)
    Wcp = jnp.zeros((HC + DE, 128), F32).at[:, 0].set(Wc[:, 0])
    bc2 = bc.reshape(1, 1)
    zrows = jnp.zeros((RPT, AW), F32)
    nb4 = node_batch.astype(I32).reshape(T, N // BMN, 1, BMN)
    eb4 = edge_batch.astype(I32).reshape(T, E // BE, 1, BE)

    xl1 = _mm(x_pkg[None], c1_Wl)
    xr1 = _mm(x_tgt, c1_Wr)
    pkg = _pkg_proj(x_pkg, proj_W, proj_b, ln1p_g, ln1p_b)
    xl2 = _mm(pkg[None], c2_Wl)
    xl1f = xl1.reshape(T * N, HC)
    xr1f = xr1.reshape(T * N, HC)
    xl2f = xl2.reshape(T * N, HC)

    def reord_g(a):
        TT = a.shape[0]
        return a.reshape(TT, NW, NCG, CH).transpose(1, 0, 2, 3).reshape(
            NW, TT * NCG, CH)

    def reord_s(a):
        TT = a.shape[0]
        return a.reshape(TT, 16, NCS, CH).transpose(1, 0, 2, 3).reshape(
            16, TT * NCS, CH)

    groups = [(i, i + TG) for i in range(0, T, TG)]
    toff_g = (jnp.arange(TG, dtype=I32) * N)[:, None]

    sir_g = [reord_g(si_off[g0:g1]) for g0, g1 in groups]
    dir_g = [reord_g(di_off[g0:g1]) for g0, g1 in groups]
    di3_g = [reord_s(dstp[g0:g1]) for g0, g1 in groups]
    dil_g = [reord_g(dstp[g0:g1] + toff_g) for g0, g1 in groups]

    h1_parts = []
    for i, (g0, g1) in enumerate(groups):
        xle, xre = _sc_gather(xl1f, xr1f, sir_g[i], dir_g[i])
        lg, mx = _logits(xle, xre, attm1[g0:g1])
        vaug = _exval(xle, lg, mx)
        accout = _sc_scatter(vaug, di3_g[i], zrows)
        h1_parts.append(_post(accout, ln1_g[g0:g1], ln1_b[g0:g1]))

    h2_parts, s_parts, smx_parts = [], [], []
    for i, (g0, g1) in enumerate(groups):
        xr2 = _mm(h1_parts[i], c2_Wr[g0:g1])
        xle2, xre2 = _sc_gather(xl2f, xr2.reshape(TG * N, HC),
                                sir_g[i], dil_g[i])
        lg2, mx2 = _logits(xle2, xre2, attm2[g0:g1])
        vaug2 = _exval(xle2, lg2, mx2)
        accout2 = _sc_scatter(vaug2, di3_g[i], zrows)
        h2g, sg, smxg = _post(accout2, ln2_g[g0:g1], ln2_b[g0:g1], wn_w)
        h2_parts.append(h2g)
        s_parts.append(sg)
        smx_parts.append(smxg)

    h2 = jnp.concatenate(h2_parts, axis=0)
    s = jnp.concatenate(s_parts, axis=0)
    smx = jnp.concatenate(smx_parts, axis=0)

    Pn = _npool(h2, s, smx, nb4)
    se, semx = _escore(edge_attr, we_w)
    Pe = _epool(edge_attr, se, semx, eb4)
    out = _final(Pn, Pe, Wcp, bc2)
    return out[:, 0]
